# baseline (speedup 1.0000x reference)
import functools

import jax
import jax.numpy as jnp
from jax import lax
from jax.experimental import pallas as pl
from jax.experimental.pallas import tpu as pltpu
from jax.experimental.pallas import tpu_sc as plsc

_B = 16384
_D = 128
_NC = 2
_NS = 16
_NW = _NC * _NS
_BW = _B // _NW
_C = 128
_NCHUNK = _BW // _C
_G = _C // 16


def _body(dflat_hbm, utab_hbm, itab_hbm, out_hbm,
          dbuf, uidx, iidx, ubuf, ibuf, tbuf, outv, sem0, sem1):
    wid = lax.axis_index("s") * _NC + lax.axis_index("c")
    base = pl.multiple_of(wid * _BW, _BW)
    base2 = pl.multiple_of(wid * (2 * _BW), 2 * _BW)

    pltpu.sync_copy(dflat_hbm.at[pl.ds(base2, 2 * _BW)], dbuf)
    lane = lax.iota(jnp.int32, 16)
    ev = lane + lane
    for j in range(_BW // 16):
        uidx[pl.ds(16 * j, 16)] = plsc.load_gather(dbuf, [ev + 32 * j])
        iidx[pl.ds(16 * j, 16)] = plsc.load_gather(dbuf, [ev + (32 * j + 1)])

    sems = (sem0, sem1)
    bufs = (ubuf, ibuf)

    def mk(c, s):
        off = pl.multiple_of(c * _C, _C)
        cu = pltpu.make_async_copy(
            utab_hbm.at[uidx.at[pl.ds(off, _C)]], ubuf.at[s], sems[s])
        ci = pltpu.make_async_copy(
            itab_hbm.at[iidx.at[pl.ds(off, _C)]], ibuf.at[s], sems[s])
        return cu, ci

    def start(c, s):
        cu, ci = mk(c, s)
        cu.start()
        ci.start()

    col_base = lane * 16

    def compute(c, s):
        ub = ubuf.at[s]
        ib = ibuf.at[s]

        def dot_row(row):
            a = ub[row, pl.ds(0, 16)] * ib[row, pl.ds(0, 16)]
            b = ub[row, pl.ds(16, 16)] * ib[row, pl.ds(16, 16)]
            for k in range(2, 8, 2):
                a = a + ub[row, pl.ds(16 * k, 16)] * ib[row, pl.ds(16 * k, 16)]
                b = b + ub[row, pl.ds(16 * (k + 1), 16)] * ib[row, pl.ds(16 * (k + 1), 16)]
            return a + b

        def group(g, carry):
            for p in range(0, 16, 4):
                acc0 = dot_row(g * 16 + p)
                acc1 = dot_row(g * 16 + p + 1)
                acc2 = dot_row(g * 16 + p + 2)
                acc3 = dot_row(g * 16 + p + 3)
                tbuf[pl.ds(16 * p, 16)] = acc0
                tbuf[pl.ds(16 * (p + 1), 16)] = acc1
                tbuf[pl.ds(16 * (p + 2), 16)] = acc2
                tbuf[pl.ds(16 * (p + 3), 16)] = acc3
            d0 = plsc.load_gather(tbuf, [col_base])
            d1 = plsc.load_gather(tbuf, [col_base + 1])
            d2 = plsc.load_gather(tbuf, [col_base + 2])
            d3 = plsc.load_gather(tbuf, [col_base + 3])
            for l in range(4, 16, 4):
                d0 = d0 + plsc.load_gather(tbuf, [col_base + l])
                d1 = d1 + plsc.load_gather(tbuf, [col_base + l + 1])
                d2 = d2 + plsc.load_gather(tbuf, [col_base + l + 2])
                d3 = d3 + plsc.load_gather(tbuf, [col_base + l + 3])
            dots = (d0 + d1) + (d2 + d3)
            res = 1.0 / (1.0 + jnp.exp(-dots))
            off = pl.multiple_of(c * _C + g * 16, 16)
            outv[pl.ds(off, 16)] = res
            return carry

        lax.fori_loop(0, _G, group, 0)

    start(0, 0)
    for c in range(_NCHUNK):
        s = c % 2
        if c + 1 < _NCHUNK:
            start(c + 1, 1 - s)
        cu, ci = mk(c, s)
        cu.wait()
        ci.wait()
        compute(c, s)

    pltpu.sync_copy(outv, out_hbm.at[pl.ds(base, _BW)])


@functools.partial(
    pl.kernel,
    out_type=jax.ShapeDtypeStruct((_B,), jnp.float32),
    mesh=plsc.VectorSubcoreMesh(
        core_axis_name="c", subcore_axis_name="s",
        num_cores=_NC, num_subcores=_NS),
    compiler_params=pltpu.CompilerParams(
        needs_layout_passes=False,
        skip_device_barrier=True,
        disable_bounds_checks=True,
        disable_semaphore_checks=True,
    ),
    scratch_types=[
        pltpu.VMEM((2 * _BW,), jnp.int32),
        pltpu.VMEM((_BW,), jnp.int32),
        pltpu.VMEM((_BW,), jnp.int32),
        pltpu.VMEM((2, _C, _D), jnp.float32),
        pltpu.VMEM((2, _C, _D), jnp.float32),
        pltpu.VMEM((256,), jnp.float32),
        pltpu.VMEM((_BW,), jnp.float32),
        pltpu.SemaphoreType.DMA,
        pltpu.SemaphoreType.DMA,
    ],
)
def _ultragcn_sc(dflat_hbm, utab_hbm, itab_hbm, out_hbm,
                 dbuf, uidx, iidx, ubuf, ibuf, tbuf, outv, sem0, sem1):
    _body(dflat_hbm, utab_hbm, itab_hbm, out_hbm,
          dbuf, uidx, iidx, ubuf, ibuf, tbuf, outv, sem0, sem1)


def kernel(data, user_table, item_table):
    return _ultragcn_sc(data.reshape(-1), user_table, item_table)


# speedup vs baseline: 1.2643x; 1.2643x over previous
import functools

import jax
import jax.numpy as jnp
from jax import lax
from jax.experimental import pallas as pl
from jax.experimental.pallas import tpu as pltpu
from jax.experimental.pallas import tpu_sc as plsc

_B = 16384
_D = 128
_NC = 2
_NS = 16
_NW = _NC * _NS
_BW = _B // _NW
_C = 128
_NCHUNK = _BW // _C
_G = _C // 16


def _body(users_hbm, items_hbm, utab_hbm, itab_hbm, out_hbm,
          uidx, iidx, ubuf, ibuf, tbuf, outv, sem0, sem1):
    wid = lax.axis_index("s") * _NC + lax.axis_index("c")
    base = pl.multiple_of(wid * _BW, _BW)

    pltpu.sync_copy(users_hbm.at[pl.ds(base, _BW)], uidx)
    pltpu.sync_copy(items_hbm.at[pl.ds(base, _BW)], iidx)
    lane = lax.iota(jnp.int32, 16)

    sems = (sem0, sem1)
    bufs = (ubuf, ibuf)

    def mk(c, s):
        off = pl.multiple_of(c * _C, _C)
        cu = pltpu.make_async_copy(
            utab_hbm.at[uidx.at[pl.ds(off, _C)]], ubuf.at[s], sems[s])
        ci = pltpu.make_async_copy(
            itab_hbm.at[iidx.at[pl.ds(off, _C)]], ibuf.at[s], sems[s])
        return cu, ci

    def start(c, s):
        cu, ci = mk(c, s)
        cu.start()
        ci.start()

    col_base = lane * 16

    def compute(c, s):
        ub = ubuf.at[s]
        ib = ibuf.at[s]

        def dot_row(row):
            a = ub[row, pl.ds(0, 16)] * ib[row, pl.ds(0, 16)]
            b = ub[row, pl.ds(16, 16)] * ib[row, pl.ds(16, 16)]
            for k in range(2, 8, 2):
                a = a + ub[row, pl.ds(16 * k, 16)] * ib[row, pl.ds(16 * k, 16)]
                b = b + ub[row, pl.ds(16 * (k + 1), 16)] * ib[row, pl.ds(16 * (k + 1), 16)]
            return a + b

        def group(g, carry):
            for p in range(0, 16, 4):
                acc0 = dot_row(g * 16 + p)
                acc1 = dot_row(g * 16 + p + 1)
                acc2 = dot_row(g * 16 + p + 2)
                acc3 = dot_row(g * 16 + p + 3)
                tbuf[pl.ds(16 * p, 16)] = acc0
                tbuf[pl.ds(16 * (p + 1), 16)] = acc1
                tbuf[pl.ds(16 * (p + 2), 16)] = acc2
                tbuf[pl.ds(16 * (p + 3), 16)] = acc3
            d0 = plsc.load_gather(tbuf, [col_base])
            d1 = plsc.load_gather(tbuf, [col_base + 1])
            d2 = plsc.load_gather(tbuf, [col_base + 2])
            d3 = plsc.load_gather(tbuf, [col_base + 3])
            for l in range(4, 16, 4):
                d0 = d0 + plsc.load_gather(tbuf, [col_base + l])
                d1 = d1 + plsc.load_gather(tbuf, [col_base + l + 1])
                d2 = d2 + plsc.load_gather(tbuf, [col_base + l + 2])
                d3 = d3 + plsc.load_gather(tbuf, [col_base + l + 3])
            dots = (d0 + d1) + (d2 + d3)
            res = 1.0 / (1.0 + jnp.exp(-dots))
            off = pl.multiple_of(c * _C + g * 16, 16)
            outv[pl.ds(off, 16)] = res
            return carry

        lax.fori_loop(0, _G, group, 0)

    start(0, 0)
    for c in range(_NCHUNK):
        s = c % 2
        if c + 1 < _NCHUNK:
            start(c + 1, 1 - s)
        cu, ci = mk(c, s)
        cu.wait()
        ci.wait()
        compute(c, s)

    pltpu.sync_copy(outv, out_hbm.at[pl.ds(base, _BW)])


@functools.partial(
    pl.kernel,
    out_type=jax.ShapeDtypeStruct((_B,), jnp.float32),
    mesh=plsc.VectorSubcoreMesh(
        core_axis_name="c", subcore_axis_name="s",
        num_cores=_NC, num_subcores=_NS),
    compiler_params=pltpu.CompilerParams(
        needs_layout_passes=False,
        skip_device_barrier=True,
        disable_bounds_checks=True,
        disable_semaphore_checks=True,
    ),
    scratch_types=[
        pltpu.VMEM((_BW,), jnp.int32),
        pltpu.VMEM((_BW,), jnp.int32),
        pltpu.VMEM((2, _C, _D), jnp.float32),
        pltpu.VMEM((2, _C, _D), jnp.float32),
        pltpu.VMEM((256,), jnp.float32),
        pltpu.VMEM((_BW,), jnp.float32),
        pltpu.SemaphoreType.DMA,
        pltpu.SemaphoreType.DMA,
    ],
)
def _ultragcn_sc(users_hbm, items_hbm, utab_hbm, itab_hbm, out_hbm,
                 uidx, iidx, ubuf, ibuf, tbuf, outv, sem0, sem1):
    _body(users_hbm, items_hbm, utab_hbm, itab_hbm, out_hbm,
          uidx, iidx, ubuf, ibuf, tbuf, outv, sem0, sem1)


def kernel(data, user_table, item_table):
    return _ultragcn_sc(data[:, 0], data[:, 1], user_table, item_table)


# speedup vs baseline: 1.3047x; 1.0319x over previous
import functools

import jax
import jax.numpy as jnp
from jax import lax
from jax.experimental import pallas as pl
from jax.experimental.pallas import tpu as pltpu
from jax.experimental.pallas import tpu_sc as plsc

_B = 16384
_D = 128
_NC = 2
_NS = 16
_NW = _NC * _NS
_BW = _B // _NW
_C = 128
_NCHUNK = _BW // _C
_G = _C // 16


def _body(users_hbm, items_hbm, utab_hbm, itab_hbm, out_hbm,
          uidx, iidx, ubuf, ibuf, tbuf, outv, sem0, sem1, sem2):
    wid = lax.axis_index("s") * _NC + lax.axis_index("c")
    base = pl.multiple_of(wid * _BW, _BW)

    lane = lax.iota(jnp.int32, 16)

    sems = (sem0, sem1)
    bufs = (ubuf, ibuf)

    def mk(c, s):
        off = pl.multiple_of(c * _C, _C)
        cu = pltpu.make_async_copy(
            utab_hbm.at[uidx.at[pl.ds(off, _C)]], ubuf.at[s], sems[s])
        ci = pltpu.make_async_copy(
            itab_hbm.at[iidx.at[pl.ds(off, _C)]], ibuf.at[s], sems[s])
        return cu, ci

    def start(c, s):
        cu, ci = mk(c, s)
        cu.start()
        ci.start()

    col_base = lane * 16

    def compute(c, s):
        ub = ubuf.at[s]
        ib = ibuf.at[s]

        def dot_row(row):
            a = ub[row, pl.ds(0, 16)] * ib[row, pl.ds(0, 16)]
            b = ub[row, pl.ds(16, 16)] * ib[row, pl.ds(16, 16)]
            for k in range(2, 8, 2):
                a = a + ub[row, pl.ds(16 * k, 16)] * ib[row, pl.ds(16 * k, 16)]
                b = b + ub[row, pl.ds(16 * (k + 1), 16)] * ib[row, pl.ds(16 * (k + 1), 16)]
            return a + b

        def group(g, carry):
            for p in range(0, 16, 4):
                acc0 = dot_row(g * 16 + p)
                acc1 = dot_row(g * 16 + p + 1)
                acc2 = dot_row(g * 16 + p + 2)
                acc3 = dot_row(g * 16 + p + 3)
                tbuf[pl.ds(16 * p, 16)] = acc0
                tbuf[pl.ds(16 * (p + 1), 16)] = acc1
                tbuf[pl.ds(16 * (p + 2), 16)] = acc2
                tbuf[pl.ds(16 * (p + 3), 16)] = acc3
            d0 = plsc.load_gather(tbuf, [col_base])
            d1 = plsc.load_gather(tbuf, [col_base + 1])
            d2 = plsc.load_gather(tbuf, [col_base + 2])
            d3 = plsc.load_gather(tbuf, [col_base + 3])
            for l in range(4, 16, 4):
                d0 = d0 + plsc.load_gather(tbuf, [col_base + l])
                d1 = d1 + plsc.load_gather(tbuf, [col_base + l + 1])
                d2 = d2 + plsc.load_gather(tbuf, [col_base + l + 2])
                d3 = d3 + plsc.load_gather(tbuf, [col_base + l + 3])
            dots = (d0 + d1) + (d2 + d3)
            res = 1.0 / (1.0 + jnp.exp(-dots))
            off = pl.multiple_of(c * _C + g * 16, 16)
            outv[pl.ds(off, 16)] = res
            return carry

        lax.fori_loop(0, _G, group, 0)

    iu0 = pltpu.make_async_copy(
        users_hbm.at[pl.ds(base, _C)], uidx.at[pl.ds(0, _C)], sem2)
    ii0 = pltpu.make_async_copy(
        items_hbm.at[pl.ds(base, _C)], iidx.at[pl.ds(0, _C)], sem2)
    iu0.start()
    ii0.start()
    iu0.wait()
    ii0.wait()
    start(0, 0)

    rbase = pl.multiple_of(base + _C, _C)
    iur = pltpu.make_async_copy(
        users_hbm.at[pl.ds(rbase, _BW - _C)], uidx.at[pl.ds(_C, _BW - _C)], sem2)
    iir = pltpu.make_async_copy(
        items_hbm.at[pl.ds(rbase, _BW - _C)], iidx.at[pl.ds(_C, _BW - _C)], sem2)
    iur.start()
    iir.start()
    iur.wait()
    iir.wait()
    start(1, 1)

    owbs = []
    for c in range(_NCHUNK):
        s = c % 2
        cu, ci = mk(c, s)
        cu.wait()
        ci.wait()
        compute(c, s)
        if c + 2 < _NCHUNK:
            start(c + 2, s)
        off = pl.multiple_of(c * _C, _C)
        ow = pltpu.make_async_copy(
            outv.at[pl.ds(off, _C)], out_hbm.at[pl.ds(base + off, _C)], sem2)
        ow.start()
        owbs.append(ow)
    for ow in owbs:
        ow.wait()


@functools.partial(
    pl.kernel,
    out_type=jax.ShapeDtypeStruct((_B,), jnp.float32),
    mesh=plsc.VectorSubcoreMesh(
        core_axis_name="c", subcore_axis_name="s",
        num_cores=_NC, num_subcores=_NS),
    compiler_params=pltpu.CompilerParams(
        needs_layout_passes=False,
        skip_device_barrier=True,
        disable_bounds_checks=True,
        disable_semaphore_checks=True,
    ),
    scratch_types=[
        pltpu.VMEM((_BW,), jnp.int32),
        pltpu.VMEM((_BW,), jnp.int32),
        pltpu.VMEM((2, _C, _D), jnp.float32),
        pltpu.VMEM((2, _C, _D), jnp.float32),
        pltpu.VMEM((256,), jnp.float32),
        pltpu.VMEM((_BW,), jnp.float32),
        pltpu.SemaphoreType.DMA,
        pltpu.SemaphoreType.DMA,
        pltpu.SemaphoreType.DMA,
    ],
)
def _ultragcn_sc(users_hbm, items_hbm, utab_hbm, itab_hbm, out_hbm,
                 uidx, iidx, ubuf, ibuf, tbuf, outv, sem0, sem1, sem2):
    _body(users_hbm, items_hbm, utab_hbm, itab_hbm, out_hbm,
          uidx, iidx, ubuf, ibuf, tbuf, outv, sem0, sem1, sem2)


def kernel(data, user_table, item_table):
    return _ultragcn_sc(data[:, 0], data[:, 1], user_table, item_table)
